# baseline (speedup 1.0000x reference)
import functools

import jax
import jax.numpy as jnp
from jax import lax
from jax.experimental import pallas as pl
from jax.experimental.pallas import tpu as pltpu
from jax.experimental.pallas import tpu_sc as plsc

DIM = 64
B = 16384
L = 50
N_SIDE = B * L
Q_SIDE = N_SIDE // 2
Q_TOT = 2 * Q_SIDE

_SC_INFO = plsc.get_sparse_core_info()
NC = _SC_INFO.num_cores
NS = _SC_INFO.num_subcores
NW = NC * NS
Q_PER_W = Q_SIDE // NW
QCHUNK = 256
N_CHUNKS = Q_PER_W // QCHUNK

MM_BLK = 4096
MM_J = B // MM_BLK
HALF = MM_BLK // 2
QL = B // 2


def _detile_body(x2_hbm, y2_hbm, out_hbm, sem):
    c = lax.axis_index("c")
    s = lax.axis_index("s")
    wid = s * NC + c
    side = wid // NS
    r0 = (wid % NS) * 4
    srcs = (x2_hbm, y2_hbm)
    for p in range(2):
        @pl.when(side == p)
        def _():
            def row_body(r, carry):
                pltpu.sync_copy(
                    srcs[p].at[r],
                    out_hbm.at[p, pl.ds(pl.multiple_of(r * B, 8), B)])
                return carry
            lax.fori_loop(r0, jnp.minimum(r0 + 4, L), row_body, 0)


_detile = functools.partial(
    pl.kernel,
    out_type=jax.ShapeDtypeStruct((2, N_SIDE), jnp.int32),
    mesh=plsc.VectorSubcoreMesh(core_axis_name="c", subcore_axis_name="s"),
    scratch_types=[pltpu.SemaphoreType.DMA],
)(_detile_body)


def _gather_body(idx_hbm, semb_hbm, temb_hbm, out_hbm,
                 idxa0_v, idxb0_v, idxa1_v, idxb1_v,
                 rowsa0_v, rowsb0_v, rowsa1_v, rowsb1_v, gsem, wsem):
    c = lax.axis_index("c")
    s = lax.axis_index("s")
    wid = s * NC + c
    base = wid * Q_PER_W
    bufs = ((idxa0_v, idxb0_v, rowsa0_v, rowsb0_v),
            (idxa1_v, idxb1_v, rowsa1_v, rowsb1_v))

    def one_side(side_qbase, side, table_hbm):
        def load_and_gather(j, buf):
            ia, ib, ra, rb = buf
            q0 = base + j * QCHUNK
            rem = q0 % QL
            na = pl.multiple_of(
                (q0 - rem) * 2 + (rem - rem % HALF) * 2 + rem % HALF, QCHUNK)
            pltpu.sync_copy(idx_hbm.at[side, pl.ds(na, QCHUNK)], ia)
            pltpu.sync_copy(idx_hbm.at[side, pl.ds(na + HALF, QCHUNK)], ib)
            pltpu.async_copy(table_hbm.at[ia], ra, gsem)
            pltpu.async_copy(table_hbm.at[ib], rb, gsem)

        def gwait(buf):
            pltpu.make_async_copy(table_hbm.at[buf[0]], buf[2], gsem).wait()
            pltpu.make_async_copy(table_hbm.at[buf[1]], buf[3], gsem).wait()

        def wstart(j, buf):
            qo = pl.multiple_of(side_qbase + base + j * QCHUNK, QCHUNK)
            pltpu.async_copy(
                buf[2], out_hbm.at[pl.ds(qo, QCHUNK), pl.ds(0, DIM)], wsem)
            pltpu.async_copy(
                buf[3], out_hbm.at[pl.ds(qo, QCHUNK), pl.ds(DIM, DIM)], wsem)

        def wwait(buf):
            pltpu.make_async_copy(
                buf[2], out_hbm.at[pl.ds(side_qbase, QCHUNK), pl.ds(0, DIM)],
                wsem).wait()
            pltpu.make_async_copy(
                buf[3], out_hbm.at[pl.ds(side_qbase, QCHUNK), pl.ds(DIM, DIM)],
                wsem).wait()

        load_and_gather(0, bufs[0])

        def body(t, carry):
            load_idx_next = 2 * t + 1
            jax.lax.cond(t > 0, lambda: wwait(bufs[1]), lambda: None)
            load_and_gather(load_idx_next, bufs[1])
            gwait(bufs[0])
            wstart(2 * t, bufs[0])

            def do_next():
                wwait(bufs[0])
                load_and_gather(2 * t + 2, bufs[0])
            jax.lax.cond(t < N_CHUNKS // 2 - 1, do_next, lambda: None)
            gwait(bufs[1])
            wstart(2 * t + 1, bufs[1])
            return carry
        lax.fori_loop(0, N_CHUNKS // 2, body, 0)
        wwait(bufs[0])
        wwait(bufs[1])

    one_side(0, 0, semb_hbm)
    one_side(Q_SIDE, 1, temb_hbm)


_gather = functools.partial(
    pl.kernel,
    out_type=jax.ShapeDtypeStruct((Q_TOT, 2 * DIM), jnp.float32),
    mesh=plsc.VectorSubcoreMesh(core_axis_name="c", subcore_axis_name="s"),
    scratch_types=[
        pltpu.VMEM((QCHUNK,), jnp.int32),
        pltpu.VMEM((QCHUNK,), jnp.int32),
        pltpu.VMEM((QCHUNK,), jnp.int32),
        pltpu.VMEM((QCHUNK,), jnp.int32),
        pltpu.VMEM((QCHUNK, DIM), jnp.float32),
        pltpu.VMEM((QCHUNK, DIM), jnp.float32),
        pltpu.VMEM((QCHUNK, DIM), jnp.float32),
        pltpu.VMEM((QCHUNK, DIM), jnp.float32),
        pltpu.SemaphoreType.DMA,
        pltpu.SemaphoreType.DMA,
    ],
    compiler_params=pltpu.CompilerParams(use_tc_tiling_on_sc=False),
)(_gather_body)


def _mm_body(g2_ref, w2_ref, o_ref):
    r = lax.dot_general(
        w2_ref[...], g2_ref[...],
        (((1,), (1,)), ((), ())),
        preferred_element_type=jnp.float32,
    )
    o_ref[0, :, 0:HALF] = r[0:DIM]
    o_ref[0, :, HALF:MM_BLK] = r[DIM:2 * DIM]


def _project(g2, W2):
    return pl.pallas_call(
        _mm_body,
        grid=(2 * L, MM_J),
        in_specs=[
            pl.BlockSpec((HALF, 2 * DIM), lambda i, j: (i * MM_J + j, 0)),
            pl.BlockSpec((2 * DIM, 2 * DIM), lambda i, j: (0, 0)),
        ],
        out_specs=pl.BlockSpec((1, DIM, MM_BLK), lambda i, j: (i, 0, j)),
        out_shape=jax.ShapeDtypeStruct((2 * L, DIM, B), jnp.float32),
    )(g2, W2)


def kernel(W_m, semb_table, temb_table, x_idx, y_idx):
    idx_flat = _detile(x_idx.T.astype(jnp.int32), y_idx.T.astype(jnp.int32))
    g2 = _gather(idx_flat, semb_table, temb_table)
    zero = jnp.zeros((DIM, DIM), jnp.float32)
    W2 = jnp.concatenate(
        [jnp.concatenate([W_m, zero], axis=1),
         jnp.concatenate([zero, W_m], axis=1)], axis=0)
    z = _project(g2, W2)
    return z.reshape(2, L, DIM, B).transpose(0, 3, 1, 2)


# speedup vs baseline: 1.1088x; 1.1088x over previous
import functools

import jax
import jax.numpy as jnp
from jax import lax
from jax.experimental import pallas as pl
from jax.experimental.pallas import tpu as pltpu
from jax.experimental.pallas import tpu_sc as plsc

DIM = 64
B = 16384
L = 50
N_SIDE = B * L
Q_SIDE = N_SIDE // 2
Q_TOT = 2 * Q_SIDE

_SC_INFO = plsc.get_sparse_core_info()
NC = _SC_INFO.num_cores
NS = _SC_INFO.num_subcores
NW = NC * NS
Q_PER_W = Q_SIDE // NW
QCHUNK = 256
N_CHUNKS = Q_PER_W // QCHUNK

MM_BLK = 4096
MM_J = B // MM_BLK
HALF = MM_BLK // 2
QL = B // 2


def _gather_body(xidx_hbm, yidx_hbm, semb_hbm, temb_hbm, out_hbm,
                 idxa0_v, idxb0_v, idxa1_v, idxb1_v,
                 rowsa0_v, rowsb0_v, rowsa1_v, rowsb1_v, gsem, wsem):
    c = lax.axis_index("c")
    s = lax.axis_index("s")
    wid = s * NC + c
    base = wid * Q_PER_W
    bufs = ((idxa0_v, idxb0_v, rowsa0_v, rowsb0_v),
            (idxa1_v, idxb1_v, rowsa1_v, rowsb1_v))

    def one_side(side_qbase, idx_hbm, table_hbm):
        def load_and_gather(j, buf):
            ia, ib, ra, rb = buf
            q0 = base + j * QCHUNK
            l = q0 // QL
            rem = q0 % QL
            boff = pl.multiple_of(
                (rem - rem % HALF) * 2 + rem % HALF, QCHUNK)
            pltpu.sync_copy(idx_hbm.at[l, pl.ds(boff, QCHUNK)], ia)
            pltpu.sync_copy(idx_hbm.at[l, pl.ds(boff + HALF, QCHUNK)], ib)
            pltpu.async_copy(table_hbm.at[ia], ra, gsem)
            pltpu.async_copy(table_hbm.at[ib], rb, gsem)

        def gwait(buf):
            pltpu.make_async_copy(table_hbm.at[buf[0]], buf[2], gsem).wait()
            pltpu.make_async_copy(table_hbm.at[buf[1]], buf[3], gsem).wait()

        def wstart(j, buf):
            qo = pl.multiple_of(side_qbase + base + j * QCHUNK, QCHUNK)
            pltpu.async_copy(
                buf[2], out_hbm.at[pl.ds(qo, QCHUNK), pl.ds(0, DIM)], wsem)
            pltpu.async_copy(
                buf[3], out_hbm.at[pl.ds(qo, QCHUNK), pl.ds(DIM, DIM)], wsem)

        def wwait(buf):
            pltpu.make_async_copy(
                buf[2], out_hbm.at[pl.ds(side_qbase, QCHUNK), pl.ds(0, DIM)],
                wsem).wait()
            pltpu.make_async_copy(
                buf[3], out_hbm.at[pl.ds(side_qbase, QCHUNK), pl.ds(DIM, DIM)],
                wsem).wait()

        load_and_gather(0, bufs[0])

        def body(t, carry):
            load_idx_next = 2 * t + 1
            jax.lax.cond(t > 0, lambda: wwait(bufs[1]), lambda: None)
            load_and_gather(load_idx_next, bufs[1])
            gwait(bufs[0])
            wstart(2 * t, bufs[0])

            def do_next():
                wwait(bufs[0])
                load_and_gather(2 * t + 2, bufs[0])
            jax.lax.cond(t < N_CHUNKS // 2 - 1, do_next, lambda: None)
            gwait(bufs[1])
            wstart(2 * t + 1, bufs[1])
            return carry
        lax.fori_loop(0, N_CHUNKS // 2, body, 0)
        wwait(bufs[0])
        wwait(bufs[1])

    one_side(0, xidx_hbm, semb_hbm)
    one_side(Q_SIDE, yidx_hbm, temb_hbm)


_gather = functools.partial(
    pl.kernel,
    out_type=jax.ShapeDtypeStruct((Q_TOT, 2 * DIM), jnp.float32),
    mesh=plsc.VectorSubcoreMesh(core_axis_name="c", subcore_axis_name="s"),
    scratch_types=[
        pltpu.VMEM((QCHUNK,), jnp.int32),
        pltpu.VMEM((QCHUNK,), jnp.int32),
        pltpu.VMEM((QCHUNK,), jnp.int32),
        pltpu.VMEM((QCHUNK,), jnp.int32),
        pltpu.VMEM((QCHUNK, DIM), jnp.float32),
        pltpu.VMEM((QCHUNK, DIM), jnp.float32),
        pltpu.VMEM((QCHUNK, DIM), jnp.float32),
        pltpu.VMEM((QCHUNK, DIM), jnp.float32),
        pltpu.SemaphoreType.DMA,
        pltpu.SemaphoreType.DMA,
    ],
    compiler_params=pltpu.CompilerParams(use_tc_tiling_on_sc=False),
)(_gather_body)


def _mm_body(g2_ref, w2_ref, o_ref):
    r = lax.dot_general(
        w2_ref[...], g2_ref[...],
        (((1,), (1,)), ((), ())),
        preferred_element_type=jnp.float32,
    )
    o_ref[0, :, 0:HALF] = r[0:DIM]
    o_ref[0, :, HALF:MM_BLK] = r[DIM:2 * DIM]


def _project(g2, W2):
    return pl.pallas_call(
        _mm_body,
        grid=(2 * L, MM_J),
        in_specs=[
            pl.BlockSpec((HALF, 2 * DIM), lambda i, j: (i * MM_J + j, 0)),
            pl.BlockSpec((2 * DIM, 2 * DIM), lambda i, j: (0, 0)),
        ],
        out_specs=pl.BlockSpec((1, DIM, MM_BLK), lambda i, j: (i, 0, j)),
        out_shape=jax.ShapeDtypeStruct((2 * L, DIM, B), jnp.float32),
    )(g2, W2)


def kernel(W_m, semb_table, temb_table, x_idx, y_idx):
    g2 = _gather(x_idx.T.astype(jnp.int32), y_idx.T.astype(jnp.int32),
                 semb_table, temb_table)
    zero = jnp.zeros((DIM, DIM), jnp.float32)
    W2 = jnp.concatenate(
        [jnp.concatenate([W_m, zero], axis=1),
         jnp.concatenate([zero, W_m], axis=1)], axis=0)
    z = _project(g2, W2)
    return z.reshape(2, L, DIM, B).transpose(0, 3, 1, 2)


# speedup vs baseline: 1.1912x; 1.0743x over previous
import functools

import jax
import jax.numpy as jnp
from jax import lax
from jax.experimental import pallas as pl
from jax.experimental.pallas import tpu as pltpu
from jax.experimental.pallas import tpu_sc as plsc

DIM = 64
B = 16384
L = 50
N_SIDE = B * L
Q_SIDE = N_SIDE // 2
Q_TOT = 2 * Q_SIDE

_SC_INFO = plsc.get_sparse_core_info()
NC = _SC_INFO.num_cores
NS = _SC_INFO.num_subcores
NW = NC * NS
Q_PER_W = Q_SIDE // NW
QCHUNK = 256
N_CHUNKS = Q_PER_W // QCHUNK

MM_BLK = 8192
MM_J = B // MM_BLK
HALF = MM_BLK // 2
QL = B // 2


def _gather_body(xidx_hbm, yidx_hbm, semb_hbm, temb_hbm, out_hbm,
                 idxa0_v, idxb0_v, idxa1_v, idxb1_v,
                 rowsa0_v, rowsb0_v, rowsa1_v, rowsb1_v, gsem, wsem):
    c = lax.axis_index("c")
    s = lax.axis_index("s")
    wid = s * NC + c
    base = wid * Q_PER_W
    bufs = ((idxa0_v, idxb0_v, rowsa0_v, rowsb0_v),
            (idxa1_v, idxb1_v, rowsa1_v, rowsb1_v))

    def one_side(side_qbase, idx_hbm, table_hbm):
        def load_and_gather(j, buf):
            ia, ib, ra, rb = buf
            q0 = base + j * QCHUNK
            l = q0 // QL
            rem = q0 % QL
            boff = pl.multiple_of(
                (rem - rem % HALF) * 2 + rem % HALF, QCHUNK)
            pltpu.sync_copy(idx_hbm.at[l, pl.ds(boff, QCHUNK)], ia)
            pltpu.sync_copy(idx_hbm.at[l, pl.ds(boff + HALF, QCHUNK)], ib)
            pltpu.async_copy(table_hbm.at[ia], ra, gsem)
            pltpu.async_copy(table_hbm.at[ib], rb, gsem)

        def gwait(buf):
            pltpu.make_async_copy(table_hbm.at[buf[0]], buf[2], gsem).wait()
            pltpu.make_async_copy(table_hbm.at[buf[1]], buf[3], gsem).wait()

        def wstart(j, buf):
            qo = pl.multiple_of(side_qbase + base + j * QCHUNK, QCHUNK)
            pltpu.async_copy(
                buf[2], out_hbm.at[pl.ds(qo, QCHUNK), pl.ds(0, DIM)], wsem)
            pltpu.async_copy(
                buf[3], out_hbm.at[pl.ds(qo, QCHUNK), pl.ds(DIM, DIM)], wsem)

        def wwait(buf):
            pltpu.make_async_copy(
                buf[2], out_hbm.at[pl.ds(side_qbase, QCHUNK), pl.ds(0, DIM)],
                wsem).wait()
            pltpu.make_async_copy(
                buf[3], out_hbm.at[pl.ds(side_qbase, QCHUNK), pl.ds(DIM, DIM)],
                wsem).wait()

        load_and_gather(0, bufs[0])

        def body(t, carry):
            load_idx_next = 2 * t + 1
            jax.lax.cond(t > 0, lambda: wwait(bufs[1]), lambda: None)
            load_and_gather(load_idx_next, bufs[1])
            gwait(bufs[0])
            wstart(2 * t, bufs[0])

            def do_next():
                wwait(bufs[0])
                load_and_gather(2 * t + 2, bufs[0])
            jax.lax.cond(t < N_CHUNKS // 2 - 1, do_next, lambda: None)
            gwait(bufs[1])
            wstart(2 * t + 1, bufs[1])
            return carry
        lax.fori_loop(0, N_CHUNKS // 2, body, 0)
        wwait(bufs[0])
        wwait(bufs[1])

    one_side(0, xidx_hbm, semb_hbm)
    one_side(Q_SIDE, yidx_hbm, temb_hbm)


_gather = functools.partial(
    pl.kernel,
    out_type=jax.ShapeDtypeStruct((Q_TOT, 2 * DIM), jnp.float32),
    mesh=plsc.VectorSubcoreMesh(core_axis_name="c", subcore_axis_name="s"),
    scratch_types=[
        pltpu.VMEM((QCHUNK,), jnp.int32),
        pltpu.VMEM((QCHUNK,), jnp.int32),
        pltpu.VMEM((QCHUNK,), jnp.int32),
        pltpu.VMEM((QCHUNK,), jnp.int32),
        pltpu.VMEM((QCHUNK, DIM), jnp.float32),
        pltpu.VMEM((QCHUNK, DIM), jnp.float32),
        pltpu.VMEM((QCHUNK, DIM), jnp.float32),
        pltpu.VMEM((QCHUNK, DIM), jnp.float32),
        pltpu.SemaphoreType.DMA,
        pltpu.SemaphoreType.DMA,
    ],
    compiler_params=pltpu.CompilerParams(use_tc_tiling_on_sc=False),
)(_gather_body)


def _mm_body(g2_ref, w2_ref, o_ref):
    r = lax.dot_general(
        w2_ref[...], g2_ref[...],
        (((1,), (1,)), ((), ())),
        preferred_element_type=jnp.float32,
    )
    o_ref[0, :, 0:HALF] = r[0:DIM]
    o_ref[0, :, HALF:MM_BLK] = r[DIM:2 * DIM]


def _project(g2, W2):
    return pl.pallas_call(
        _mm_body,
        grid=(2 * L, MM_J),
        in_specs=[
            pl.BlockSpec((HALF, 2 * DIM), lambda i, j: (i * MM_J + j, 0)),
            pl.BlockSpec((2 * DIM, 2 * DIM), lambda i, j: (0, 0)),
        ],
        out_specs=pl.BlockSpec((1, DIM, MM_BLK), lambda i, j: (i, 0, j)),
        out_shape=jax.ShapeDtypeStruct((2 * L, DIM, B), jnp.float32),
    )(g2, W2)


def kernel(W_m, semb_table, temb_table, x_idx, y_idx):
    g2 = _gather(x_idx.T.astype(jnp.int32), y_idx.T.astype(jnp.int32),
                 semb_table, temb_table)
    zero = jnp.zeros((DIM, DIM), jnp.float32)
    W2 = jnp.concatenate(
        [jnp.concatenate([W_m, zero], axis=1),
         jnp.concatenate([zero, W_m], axis=1)], axis=0)
    z = _project(g2, W2)
    return z.reshape(2, L, DIM, B).transpose(0, 3, 1, 2)
